# Initial kernel scaffold; baseline (speedup 1.0000x reference)
import jax
import jax.numpy as jnp
from jax.experimental import pallas as pl


def kernel(scale, pr):
    raise NotImplementedError("write your pallas kernel here")


# speedup vs baseline: 13.6577x; 13.6577x over previous
import jax
import jax.numpy as jnp
from jax.experimental import pallas as pl
from jax.experimental.pallas import tpu as pltpu

def _qmask_kernel(ki_ref, kf_ref, x_ref, out_ref):
    _INT_MIN = jnp.int32(-(2**31))
    _INT_MAX = jnp.int32(2**31 - 1)
    _LOW31 = jnp.int32(0x7FFFFFFF)
    x = x_ref[...]
    b = jax.lax.bitcast_convert_type(x, jnp.int32)
    key = jnp.where(b < 0, b ^ _LOW31, b)

    k_lo = ki_ref[0, 0]
    k_hi = ki_ref[0, 1]
    pr = ki_ref[0, 2]
    frac = kf_ref[0, 0]

    cnt0 = jnp.sum((key < 0).astype(jnp.int32), axis=1, keepdims=True)
    p0 = jnp.where(cnt0 <= k_lo, jnp.int32(0), _INT_MIN)

    def body(i, p):
        bit = jax.lax.shift_left(jnp.int32(1), jnp.int32(30) - i)
        cand = p + bit
        cnt = jnp.sum((key < cand).astype(jnp.int32), axis=1, keepdims=True)
        return jnp.where(cnt <= k_lo, cand, p)

    a_lo = jax.lax.fori_loop(0, 31, body, p0)

    le = jnp.sum((key <= a_lo).astype(jnp.int32), axis=1, keepdims=True)
    gt_min = jnp.min(jnp.where(key > a_lo, key, _INT_MAX), axis=1, keepdims=True)
    a_hi = jnp.where(le > k_hi, a_lo, gt_min)

    def tofloat(a):
        bits = jnp.where(a < 0, a ^ _LOW31, a)
        return jax.lax.bitcast_convert_type(bits, jnp.float32)

    v_lo = tofloat(a_lo)
    v_hi = tofloat(a_hi)
    q = v_lo * (jnp.float32(1.0) - frac) + v_hi * frac
    q = jnp.where(pr == 0, jnp.float32(jnp.inf), q)
    q = jnp.where(pr >= 10, jnp.float32(-jnp.inf), q)
    out_ref[...] = (x >= q).astype(jnp.float32)


def kernel(scale, pr):
    bs, ch, w, h = scale.shape
    n = ch * w * h
    flat = scale.reshape(bs, n)

    pr_arr = jnp.asarray(pr, jnp.int32)
    pr_f = jnp.where(pr_arr > 10, 10, pr_arr) * jnp.float32(0.1)
    pr_bis = jnp.float32(1.0) - pr_f
    idx = pr_bis * jnp.float32(n - 1)
    low = jnp.floor(idx)
    frac = jnp.clip(idx - low, 0.0, 1.0).reshape(1, 1)
    k_lo = jnp.clip(low.astype(jnp.int32), 0, n - 1)
    k_hi = jnp.clip(jnp.ceil(idx).astype(jnp.int32), 0, n - 1)
    ki = jnp.stack([k_lo, k_hi, pr_arr]).reshape(1, 3)

    out = pl.pallas_call(
        _qmask_kernel,
        out_shape=jax.ShapeDtypeStruct((bs, n), jnp.float32),
        in_specs=[
            pl.BlockSpec(memory_space=pltpu.SMEM),
            pl.BlockSpec(memory_space=pltpu.SMEM),
            pl.BlockSpec(memory_space=pltpu.VMEM),
        ],
        out_specs=pl.BlockSpec(memory_space=pltpu.VMEM),
    )(ki, frac, flat)
    return out.reshape(bs, ch, w, h)
